# baseline (speedup 1.0000x reference)
import jax
import jax.numpy as jnp
from jax import lax
from jax.experimental import pallas as pl
from jax.experimental.pallas import tpu as pltpu
from jax.experimental.pallas import tpu_sc as plsc

D_MODEL = 64
N_ROWS = 4096
SEQ = 200
B = N_ROWS * SEQ
NC, NS = 2, 16
NW = NC * NS
B_PER_W = B // NW
CH = 128
NCHUNK = B_PER_W // CH
NBUF = 4
LAG = 2
SCALE = 8.0

_mesh = plsc.VectorSubcoreMesh(core_axis_name="c", subcore_axis_name="s")


def _sc_body(idx_hbm, table_hbm, out_hbm, idx_v, r0, r1, r2, r3,
             g0, g1, g2, g3, o0, o1, o2, o3):
    rows = (r0, r1, r2, r3)
    gsem = (g0, g1, g2, g3)
    osem = (o0, o1, o2, o3)
    wid = lax.axis_index("s") * NC + lax.axis_index("c")
    base = wid * B_PER_W

    pltpu.sync_copy(idx_hbm.at[wid], idx_v)

    def out_ref(j):
        return out_hbm.at[pl.ds(base + j * CH, CH)]

    def gather(j, b):
        return pltpu.make_async_copy(table_hbm.at[idx_v.at[j]], rows[b],
                                     gsem[b])

    def out_copy(j, b):
        return pltpu.make_async_copy(rows[b], out_ref(j), osem[b])

    def scale_buf(b):
        buf = rows[b]

        @pl.loop(0, CH)
        def _(r):
            for c in range(D_MODEL // 16):
                slc = (pl.ds(r, 1), pl.ds(c * 16, 16))
                buf[slc] = buf[slc] * SCALE

    for j in range(LAG):
        gather(j, j % NBUF).start()

    @pl.loop(0, NCHUNK // NBUF)
    def _(g):
        j0 = g * NBUF
        for b in range(NBUF):
            j = j0 + b
            bp = (b + LAG) % NBUF

            @pl.when(j >= LAG)
            def _():
                out_copy(j - LAG, bp).wait()

            @pl.when(j + LAG < NCHUNK)
            def _():
                gather(j + LAG, bp).start()

            gather(j, b).wait()
            scale_buf(b)
            out_copy(j, b).start()

    for j in range(NCHUNK - LAG, NCHUNK):
        out_copy(j, j % NBUF).wait()


_sc_call = pl.kernel(
    _sc_body,
    out_type=jax.ShapeDtypeStruct((B, D_MODEL), jnp.float32),
    mesh=_mesh,
    compiler_params=pltpu.CompilerParams(use_tc_tiling_on_sc=False),
    scratch_types=[
        pltpu.VMEM((NCHUNK, CH), jnp.int32),
        pltpu.VMEM((CH, D_MODEL), jnp.float32),
        pltpu.VMEM((CH, D_MODEL), jnp.float32),
        pltpu.VMEM((CH, D_MODEL), jnp.float32),
        pltpu.VMEM((CH, D_MODEL), jnp.float32),
        pltpu.SemaphoreType.DMA,
        pltpu.SemaphoreType.DMA,
        pltpu.SemaphoreType.DMA,
        pltpu.SemaphoreType.DMA,
        pltpu.SemaphoreType.DMA,
        pltpu.SemaphoreType.DMA,
        pltpu.SemaphoreType.DMA,
        pltpu.SemaphoreType.DMA,
    ],
)


def kernel(x, table):
    xw = x.astype(jnp.int32).reshape(NW, NCHUNK, CH)
    return _sc_call(xw, table).reshape(N_ROWS, SEQ, D_MODEL)


# speedup vs baseline: 1.1261x; 1.1261x over previous
import jax
import jax.numpy as jnp
from jax import lax
from jax.experimental import pallas as pl
from jax.experimental.pallas import tpu as pltpu
from jax.experimental.pallas import tpu_sc as plsc

D_MODEL = 64
N_ROWS = 4096
SEQ = 200
B = N_ROWS * SEQ
NC, NS = 2, 16
NW = NC * NS
B_PER_W = B // NW
CH = 128
NCHUNK = B_PER_W // CH
NBUF = 4
LAG = 2
SCALE = 8.0

_mesh = plsc.VectorSubcoreMesh(core_axis_name="c", subcore_axis_name="s")


def _sc_body(idx_hbm, table_hbm, out_hbm, idx_v, r0, r1, r2, r3,
             g0, g1, g2, g3, o0, o1, o2, o3):
    rows = (r0, r1, r2, r3)
    gsem = (g0, g1, g2, g3)
    osem = (o0, o1, o2, o3)
    wid = lax.axis_index("s") * NC + lax.axis_index("c")
    base = wid * B_PER_W

    pltpu.sync_copy(idx_hbm.at[wid], idx_v)

    def out_ref(j):
        return out_hbm.at[pl.ds(base + j * CH, CH)]

    def gather(j, b):
        return pltpu.make_async_copy(table_hbm.at[idx_v.at[j]], rows[b],
                                     gsem[b])

    def out_copy(j, b):
        return pltpu.make_async_copy(rows[b], out_ref(j), osem[b])

    def scale_buf(b):
        buf = rows[b]

        @pl.loop(0, CH)
        def _(r):
            for c in range(D_MODEL // 16):
                slc = (pl.ds(r, 1), pl.ds(c * 16, 16))
                buf[slc] = buf[slc] * SCALE

    for j in range(LAG):
        gather(j, j % NBUF).start()

    @pl.loop(0, NCHUNK // NBUF)
    def _(g):
        j0 = g * NBUF
        for b in range(NBUF):
            j = j0 + b
            bp = (b + LAG) % NBUF

            @pl.when(j >= LAG)
            def _():
                out_copy(j - LAG, bp).wait()

            @pl.when(j + LAG < NCHUNK)
            def _():
                gather(j + LAG, bp).start()

            gather(j, b).wait()
            scale_buf(b)
            out_copy(j, b).start()

    for j in range(NCHUNK - LAG, NCHUNK):
        out_copy(j, j % NBUF).wait()


_sc_call = pl.kernel(
    _sc_body,
    out_type=jax.ShapeDtypeStruct((B, D_MODEL), jnp.float32),
    mesh=_mesh,
    compiler_params=pltpu.CompilerParams(use_tc_tiling_on_sc=False),
    scratch_types=[
        pltpu.VMEM((NCHUNK, CH), jnp.int32),
        pltpu.VMEM((CH, D_MODEL), jnp.float32),
        pltpu.VMEM((CH, D_MODEL), jnp.float32),
        pltpu.VMEM((CH, D_MODEL), jnp.float32),
        pltpu.VMEM((CH, D_MODEL), jnp.float32),
        pltpu.SemaphoreType.DMA,
        pltpu.SemaphoreType.DMA,
        pltpu.SemaphoreType.DMA,
        pltpu.SemaphoreType.DMA,
        pltpu.SemaphoreType.DMA,
        pltpu.SemaphoreType.DMA,
        pltpu.SemaphoreType.DMA,
        pltpu.SemaphoreType.DMA,
    ],
)


def kernel(x, table):
    xw = x.astype(jnp.int32).reshape(NW, NCHUNK, CH)
    tbl_pinned = lax.optimization_barrier(table)
    tbl2 = lax.optimization_barrier(tbl_pinned.reshape(500000, 2 * D_MODEL))
    tbl = tbl2.reshape(1000000, D_MODEL)
    out = _sc_call(xw, tbl)
    y = out.reshape(N_ROWS, SEQ * D_MODEL)
    y_t = lax.optimization_barrier(y.T)
    return y_t.reshape(SEQ, D_MODEL, N_ROWS).transpose(2, 0, 1)
